# baseline (speedup 1.0000x reference)
import functools

import jax
import jax.numpy as jnp
from jax import lax
from jax.experimental import pallas as pl
from jax.experimental.pallas import tpu as pltpu
from jax.experimental.pallas import tpu_sc as plsc

_NC = 2
_NS = 16
_NW = _NC * _NS

_K = 128
_NBUF = 4


@functools.lru_cache(maxsize=None)
def _build(C, K, D):
    mesh = plsc.VectorSubcoreMesh(core_axis_name="c", subcore_axis_name="s")
    n_rows = _NW * C * K

    scratch = [
        pltpu.VMEM((C, K), jnp.int32),
        pltpu.VMEM((_NBUF, K, D), jnp.float32),
    ]
    scratch += [pltpu.SemaphoreType.DMA] * (2 * _NBUF)

    @functools.partial(
        pl.kernel,
        mesh=mesh,
        out_type=jax.ShapeDtypeStruct((n_rows, D), jnp.float32),
        scratch_types=scratch,
        compiler_params=pltpu.CompilerParams(use_tc_tiling_on_sc=False),
    )
    def emb(idx_hbm, tab_hbm, out_hbm, idx_v, rows_v, *sems):
        gsem = sems[:_NBUF]
        ssem = sems[_NBUF:]
        wid = lax.axis_index("s") * _NC + lax.axis_index("c")
        row0 = wid * (C * K)

        pltpu.sync_copy(idx_hbm.at[wid], idx_v)

        def g_copy(j, b):
            return pltpu.make_async_copy(
                tab_hbm.at[idx_v.at[j]], rows_v.at[b], gsem[b])

        def s_copy(j, b):
            return pltpu.make_async_copy(
                rows_v.at[b], out_hbm.at[pl.ds(row0 + j * K, K)], ssem[b])

        for b in range(_NBUF):
            g_copy(b, b).start()

        def group(g, carry):
            for b in range(_NBUF):
                j = g * _NBUF + b
                g_copy(j, b).wait()
                s_copy(j, b).start()
                s_copy(j, b).wait()
                g_copy(j + _NBUF, b).start()
            return carry

        lax.fori_loop(0, C // _NBUF - 1, group, 0)

        for b in range(_NBUF):
            j = (C // _NBUF - 1) * _NBUF + b
            g_copy(j, b).wait()
            s_copy(j, b).start()
            s_copy(j, b).wait()

    return emb


def kernel(input, weight):
    B, H = input.shape
    V, D = weight.shape
    N = B * H
    per_w = N // _NW
    C = per_w // _K
    assert per_w % _K == 0 and C % _NBUF == 0 and N % _NW == 0

    idx3 = input.reshape(_NW, C, _K).astype(jnp.int32)
    out = _build(C, _K, D)(idx3, weight)
    return out.reshape(B, H, D)
